# baseline (speedup 1.0000x reference)
import functools

import jax
import jax.numpy as jnp
from jax import lax
from jax.experimental import pallas as pl
from jax.experimental.pallas import tpu as pltpu
from jax.experimental.pallas import tpu_sc as plsc

M = 65536
B = 4096
D = 512
W = 2 * D + 2
NW = 32
RW = M // NW
CH = 64

_mesh = plsc.VectorSubcoreMesh(core_axis_name="c", subcore_axis_name="s")


@functools.partial(
    pl.kernel,
    mesh=_mesh,
    out_type=jax.ShapeDtypeStruct((M, W), jnp.float32),
    scratch_types=[
        pltpu.VMEM((CH, D), jnp.float32),
        pltpu.VMEM((CH, D), jnp.float32),
        pltpu.VMEM((CH, 2), jnp.float32),
        pltpu.VMEM((16,), jnp.int32),
    ],
)
def _sc_update(ptr_hbm, mi, mg, mc, ni, ng, nc, out, bimg, bgps, bcrd, psmem):
    wid = lax.axis_index("s") * 2 + lax.axis_index("c")
    pltpu.sync_copy(ptr_hbm, psmem)
    p = psmem[...][0]
    base = wid * RW
    off = lax.rem(base - p + M, M)
    is_new = off < B

    def chunk(c, carry):
        r0 = pl.multiple_of(base + c * CH, CH)

        @pl.when(is_new)
        def _():
            s = pl.multiple_of(off + c * CH, CH)
            pltpu.sync_copy(ni.at[pl.ds(s, CH)], bimg)
            pltpu.sync_copy(ng.at[pl.ds(s, CH)], bgps)
            pltpu.sync_copy(nc.at[pl.ds(s, CH)], bcrd)

        @pl.when(jnp.logical_not(is_new))
        def _():
            pltpu.sync_copy(mi.at[pl.ds(r0, CH)], bimg)
            pltpu.sync_copy(mg.at[pl.ds(r0, CH)], bgps)
            pltpu.sync_copy(mc.at[pl.ds(r0, CH)], bcrd)

        pltpu.sync_copy(bimg, out.at[pl.ds(r0, CH), pl.ds(0, D)])
        pltpu.sync_copy(bgps, out.at[pl.ds(r0, CH), pl.ds(D, D)])
        pltpu.sync_copy(bcrd, out.at[pl.ds(r0, CH), pl.ds(2 * D, 2)])
        return carry

    lax.fori_loop(0, RW // CH, chunk, 0)


def kernel(mem_img, mem_gps, mem_coords, img_emb, gps_emb, gps_coords, ptr):
    ptr_arr = jnp.full((16,), ptr, dtype=jnp.int32)
    return _sc_update(ptr_arr, mem_img, mem_gps, mem_coords,
                      img_emb, gps_emb, gps_coords)


# speedup vs baseline: 1.1176x; 1.1176x over previous
import functools

import jax
import jax.numpy as jnp
from jax import lax
from jax.experimental import pallas as pl
from jax.experimental.pallas import tpu as pltpu
from jax.experimental.pallas import tpu_sc as plsc

M = 65536
B = 4096
D = 512
W = 2 * D + 2
NW = 32
RW = M // NW
CH = 32
NCH = RW // CH

_mesh = plsc.VectorSubcoreMesh(core_axis_name="c", subcore_axis_name="s")


@functools.partial(
    pl.kernel,
    mesh=_mesh,
    out_type=jax.ShapeDtypeStruct((M, W), jnp.float32),
    scratch_types=[
        pltpu.VMEM((2, CH, D), jnp.float32),
        pltpu.VMEM((2, CH, D), jnp.float32),
        pltpu.VMEM((2, CH, 2), jnp.float32),
        pltpu.VMEM((16,), jnp.int32),
        pltpu.SemaphoreType.DMA,
        pltpu.SemaphoreType.DMA,
        pltpu.SemaphoreType.DMA,
        pltpu.SemaphoreType.DMA,
    ],
)
def _sc_update(ptr_hbm, mi, mg, mc, ni, ng, nc, out,
               bimg, bgps, bcrd, pv, sg0, sg1, ss0, ss1):
    wid = lax.axis_index("s") * 2 + lax.axis_index("c")
    pltpu.sync_copy(ptr_hbm, pv)
    p = pv[...][0]
    base = pl.multiple_of(wid * RW, RW)
    off = pl.multiple_of(lax.rem(base - p + M, M), RW)
    is_new = off < B
    not_new = jnp.logical_not(is_new)
    sg = (sg0, sg1)
    ss = (ss0, ss1)

    def gather_start(c, b):
        @pl.when(is_new)
        def _():
            s = pl.multiple_of(off + c * CH, CH)
            pltpu.make_async_copy(ni.at[pl.ds(s, CH)], bimg.at[b], sg[b]).start()
            pltpu.make_async_copy(ng.at[pl.ds(s, CH)], bgps.at[b], sg[b]).start()
            pltpu.make_async_copy(nc.at[pl.ds(s, CH)], bcrd.at[b], sg[b]).start()

        @pl.when(not_new)
        def _():
            r = pl.multiple_of(base + c * CH, CH)
            pltpu.make_async_copy(mi.at[pl.ds(r, CH)], bimg.at[b], sg[b]).start()
            pltpu.make_async_copy(mg.at[pl.ds(r, CH)], bgps.at[b], sg[b]).start()
            pltpu.make_async_copy(mc.at[pl.ds(r, CH)], bcrd.at[b], sg[b]).start()

    def gather_wait(b):
        pltpu.make_async_copy(mi.at[pl.ds(0, CH)], bimg.at[b], sg[b]).wait()
        pltpu.make_async_copy(mg.at[pl.ds(0, CH)], bgps.at[b], sg[b]).wait()
        pltpu.make_async_copy(mc.at[pl.ds(0, CH)], bcrd.at[b], sg[b]).wait()

    def s_copies(c, b):
        r = pl.multiple_of(base + c * CH, CH)
        return (
            pltpu.make_async_copy(bimg.at[b],
                                  out.at[pl.ds(r, CH), pl.ds(0, D)], ss[b]),
            pltpu.make_async_copy(bgps.at[b],
                                  out.at[pl.ds(r, CH), pl.ds(D, D)], ss[b]),
            pltpu.make_async_copy(bcrd.at[b],
                                  out.at[pl.ds(r, CH), pl.ds(2 * D, 2)], ss[b]),
        )

    def scatter_start(c, b):
        for cp in s_copies(c, b):
            cp.start()

    def scatter_wait(c, b):
        for cp in s_copies(c, b):
            cp.wait()

    gather_start(0, 0)
    gather_start(1, 1)

    @pl.loop(0, NCH - 2, step=2)
    def _(g):
        for b in (0, 1):
            c = g + b
            gather_wait(b)
            scatter_start(c, b)
            scatter_wait(c, b)
            gather_start(c + 2, b)

    for b in (0, 1):
        c = NCH - 2 + b
        gather_wait(b)
        scatter_start(c, b)
        scatter_wait(c, b)


def kernel(mem_img, mem_gps, mem_coords, img_emb, gps_emb, gps_coords, ptr):
    ptr_arr = jnp.full((16,), ptr, dtype=jnp.int32)
    return _sc_update(ptr_arr, mem_img, mem_gps, mem_coords,
                      img_emb, gps_emb, gps_coords)


# speedup vs baseline: 1.1219x; 1.0038x over previous
import functools

import jax
import jax.numpy as jnp
from jax import lax
from jax.experimental import pallas as pl
from jax.experimental.pallas import tpu as pltpu
from jax.experimental.pallas import tpu_sc as plsc

M = 65536
B = 4096
D = 512
W = 2 * D + 2
NW = 32
RW = M // NW
CH = 32
NCH = RW // CH

_mesh = plsc.VectorSubcoreMesh(core_axis_name="c", subcore_axis_name="s")


@functools.partial(
    pl.kernel,
    mesh=_mesh,
    out_type=jax.ShapeDtypeStruct((M, W), jnp.float32),
    scratch_types=[
        pltpu.VMEM((2, CH, W), jnp.float32),
        pltpu.VMEM((16,), jnp.int32),
        pltpu.SemaphoreType.DMA,
        pltpu.SemaphoreType.DMA,
        pltpu.SemaphoreType.DMA,
        pltpu.SemaphoreType.DMA,
    ],
)
def _sc_update(ptr_hbm, mi, mg, mc, ni, ng, nc, out,
               buf, pv, sg0, sg1, ss0, ss1):
    wid = lax.axis_index("s") * 2 + lax.axis_index("c")
    pltpu.sync_copy(ptr_hbm, pv)
    p = pv[...][0]
    base = pl.multiple_of(wid * RW, RW)
    off = pl.multiple_of(lax.rem(base - p + M, M), RW)
    is_new = off < B
    not_new = jnp.logical_not(is_new)
    sg = (sg0, sg1)
    ss = (ss0, ss1)

    def gather_start(c, b):
        @pl.when(is_new)
        def _():
            s = pl.multiple_of(off + c * CH, CH)
            pltpu.make_async_copy(ni.at[pl.ds(s, CH)],
                                  buf.at[b, :, pl.ds(0, D)], sg[b]).start()
            pltpu.make_async_copy(ng.at[pl.ds(s, CH)],
                                  buf.at[b, :, pl.ds(D, D)], sg[b]).start()
            pltpu.make_async_copy(nc.at[pl.ds(s, CH)],
                                  buf.at[b, :, pl.ds(2 * D, 2)], sg[b]).start()

        @pl.when(not_new)
        def _():
            r = pl.multiple_of(base + c * CH, CH)
            pltpu.make_async_copy(mi.at[pl.ds(r, CH)],
                                  buf.at[b, :, pl.ds(0, D)], sg[b]).start()
            pltpu.make_async_copy(mg.at[pl.ds(r, CH)],
                                  buf.at[b, :, pl.ds(D, D)], sg[b]).start()
            pltpu.make_async_copy(mc.at[pl.ds(r, CH)],
                                  buf.at[b, :, pl.ds(2 * D, 2)], sg[b]).start()

    def gather_wait(b):
        pltpu.make_async_copy(mi.at[pl.ds(0, CH)],
                              buf.at[b, :, pl.ds(0, D)], sg[b]).wait()
        pltpu.make_async_copy(mg.at[pl.ds(0, CH)],
                              buf.at[b, :, pl.ds(D, D)], sg[b]).wait()
        pltpu.make_async_copy(mc.at[pl.ds(0, CH)],
                              buf.at[b, :, pl.ds(2 * D, 2)], sg[b]).wait()

    def s_copy(c, b):
        r = pl.multiple_of(base + c * CH, CH)
        return pltpu.make_async_copy(buf.at[b], out.at[pl.ds(r, CH)], ss[b])

    gather_start(0, 0)
    gather_start(1, 1)

    @pl.loop(0, NCH - 2, step=2)
    def _(g):
        for b in (0, 1):
            c = g + b
            gather_wait(b)
            s_copy(c, b).start()
            s_copy(c, b).wait()
            gather_start(c + 2, b)

    for b in (0, 1):
        c = NCH - 2 + b
        gather_wait(b)
        s_copy(c, b).start()
        s_copy(c, b).wait()


def kernel(mem_img, mem_gps, mem_coords, img_emb, gps_emb, gps_coords, ptr):
    ptr_arr = jnp.full((16,), ptr, dtype=jnp.int32)
    return _sc_update(ptr_arr, mem_img, mem_gps, mem_coords,
                      img_emb, gps_emb, gps_coords)
